# Initial kernel scaffold; baseline (speedup 1.0000x reference)
import jax
import jax.numpy as jnp
from jax.experimental import pallas as pl


def kernel(x_user, x_movie, edge_label_index):
    raise NotImplementedError("write your pallas kernel here")


# speedup vs baseline: 2.2742x; 2.2742x over previous
import functools

import jax
import jax.numpy as jnp
from jax import lax
from jax.experimental import pallas as pl
from jax.experimental.pallas import tpu as pltpu, tpu_sc as plsc

D = 128
C = 128

_info = plsc.get_sparse_core_info()
NC, NS, L = _info.num_cores, _info.num_subcores, _info.num_lanes
NW = NC * NS


def _make_sc_kernel(padded_b: int):
    n_chunks = padded_b // (NW * C)
    mesh = plsc.VectorSubcoreMesh(core_axis_name="c", subcore_axis_name="s")

    @functools.partial(
        pl.kernel,
        mesh=mesh,
        compiler_params=pltpu.CompilerParams(needs_layout_passes=False),
        out_type=jax.ShapeDtypeStruct((padded_b,), jnp.float32),
        scratch_types=[
            pltpu.VMEM((C,), jnp.int32),
            pltpu.VMEM((C,), jnp.int32),
            pltpu.VMEM((C, D), jnp.float32),
            pltpu.VMEM((C, D), jnp.float32),
            pltpu.VMEM((C,), jnp.float32),
            pltpu.SemaphoreType.DMA,
        ],
    )
    def sc_kernel(u_tbl, m_tbl, uidx_hbm, midx_hbm, out_hbm,
                  uidx_v, midx_v, u_v, m_v, out_v, sem):
        wid = lax.axis_index("s") * NC + lax.axis_index("c")

        def chunk_body(it, _):
            base = (it * NW + wid) * C
            pltpu.sync_copy(uidx_hbm.at[pl.ds(base, C)], uidx_v)
            pltpu.sync_copy(midx_hbm.at[pl.ds(base, C)], midx_v)
            cp_u = pltpu.async_copy(u_tbl.at[uidx_v], u_v, sem)
            cp_m = pltpu.async_copy(m_tbl.at[midx_v], m_v, sem)
            cp_u.wait()
            cp_m.wait()

            lanes = lax.iota(jnp.int32, L)

            def g_body(g, _):
                out_vec = jnp.zeros((L,), jnp.float32)
                for j in range(L):
                    e = g * L + j
                    acc = u_v[e, pl.ds(0, L)] * m_v[e, pl.ds(0, L)]
                    for i in range(1, D // L):
                        acc = acc + (u_v[e, pl.ds(i * L, L)]
                                     * m_v[e, pl.ds(i * L, L)])
                    out_vec = jnp.where(lanes == j, jnp.sum(acc), out_vec)
                out_v[pl.ds(pl.multiple_of(g * L, L), L)] = out_vec
                return ()

            lax.fori_loop(0, C // L, g_body, ())

            pltpu.sync_copy(out_v, out_hbm.at[pl.ds(base, C)])
            return ()

        lax.fori_loop(0, n_chunks, chunk_body, ())

    return sc_kernel


def kernel(x_user, x_movie, edge_label_index):
    eli = edge_label_index.astype(jnp.int32)
    b = eli.shape[1]
    grain = NW * C
    padded_b = ((b + grain - 1) // grain) * grain
    uidx = jnp.pad(eli[0], (0, padded_b - b))
    midx = jnp.pad(eli[1], (0, padded_b - b))
    out = _make_sc_kernel(padded_b)(x_user, x_movie, uidx, midx)
    return out[:b]
